# baseline (speedup 1.0000x reference)
import jax
import jax.numpy as jnp
from jax import lax
from jax.experimental import pallas as pl
from jax.experimental.pallas import tpu as pltpu
from jax.experimental.pallas import tpu_sc as plsc

B = 16384
C = 1000
NC = 2
NS = 16
NW = NC * NS
PW = B // NW
NB = PW // 128
L = 16


def _body(flat, y3, out3, yv, res, sem):
    wid = lax.axis_index("s") * NC + lax.axis_index("c")

    pltpu.sync_copy(y3.at[wid], yv)

    iota = lax.iota(jnp.int32, L)
    copies = []
    for c in range(NB):
        blk = (wid * NB + c) * 1024

        def _cvt(k, carry, c=c, blk=blk):
            s = pl.multiple_of(k * L, L)
            yvec = yv[c, pl.ds(s, L)]
            idx = (
                lax.shift_right_logical(yvec, 3) * 131072
                + (yvec & 7) * 128
                + (blk + k * L)
                + iota
            )
            yv[c, pl.ds(s, L)] = idx
            return carry

        lax.fori_loop(0, 128 // L, _cvt, 0, unroll=False)
        for h in range(2):
            copies.append(
                pltpu.async_copy(
                    flat.at[yv.at[c, pl.ds(h * 64, 64)]],
                    res.at[c, pl.ds(h * 64, 64)],
                    sem,
                )
            )
    for cp in copies:
        cp.wait()

    pltpu.sync_copy(res, out3.at[wid])


@jax.jit
def kernel(logits, y):
    flat = (
        logits.T.reshape(C // 8, 8, B // 128, 128)
        .transpose(0, 2, 1, 3)
        .reshape(-1)
    )
    y3 = y.astype(jnp.int32).reshape(NW, NB, 128)

    mesh = plsc.VectorSubcoreMesh(core_axis_name="c", subcore_axis_name="s")
    out3 = pl.kernel(
        _body,
        out_type=jax.ShapeDtypeStruct((NW, NB, 128), jnp.float32),
        mesh=mesh,
        compiler_params=pltpu.CompilerParams(needs_layout_passes=False),
        scratch_types=[
            pltpu.VMEM((NB, 128), jnp.int32),
            pltpu.VMEM((NB, 128), jnp.float32),
            pltpu.SemaphoreType.DMA,
        ],
    )(flat, y3)
    return out3.reshape(-1)
